# baseline (speedup 1.0000x reference)
import functools

import jax
import jax.numpy as jnp
from jax import lax
from jax.experimental import pallas as pl
from jax.experimental.pallas import tpu as pltpu
from jax.experimental.pallas import tpu_sc as plsc

MAXLEN = 200
EMB = 64
BATCH_LANES = 128
NUM_TILES = 32
NGBUF = 4


def _tok_pos_embed(x_t, token_table, pos_table, batch):
    nb = batch // BATCH_LANES
    assert nb == NUM_TILES
    mesh = plsc.VectorSubcoreMesh(core_axis_name="c", subcore_axis_name="s")

    @functools.partial(
        pl.kernel,
        out_type=jax.ShapeDtypeStruct((MAXLEN, EMB // 8, nb, 8 * BATCH_LANES),
                                      jnp.float32),
        mesh=mesh,
        compiler_params=pltpu.CompilerParams(use_tc_tiling_on_sc=False,
                                             needs_layout_passes=False),
        scratch_types=[
            pltpu.VMEM((MAXLEN, BATCH_LANES), jnp.int32),
            pltpu.VMEM((MAXLEN, EMB), jnp.float32),
        ] + [pltpu.VMEM((BATCH_LANES, EMB), jnp.float32) for _ in range(NGBUF)]
          + [pltpu.VMEM((EMB * BATCH_LANES,), jnp.float32) for _ in range(2)]
          + [pltpu.SemaphoreType.DMA for _ in range(NGBUF + 2)],
    )
    def k(x_hbm, tok_hbm, pos_hbm, out_hbm, idx_v, pos_v, *bufs_and_sems):
        gbufs = bufs_and_sems[:NGBUF]
        tbufs = bufs_and_sems[NGBUF:NGBUF + 2]
        gsems = bufs_and_sems[NGBUF + 2:2 * NGBUF + 2]
        osems = bufs_and_sems[2 * NGBUF + 2:]
        wid = lax.axis_index("s") * 2 + lax.axis_index("c")
        pltpu.sync_copy(x_hbm.at[:, pl.ds(wid * BATCH_LANES, BATCH_LANES)],
                        idx_v)
        pltpu.sync_copy(pos_hbm, pos_v)
        lane = lax.iota(jnp.int32, 16)

        def issue_gather(p, b):
            pltpu.async_copy(tok_hbm.at[idx_v.at[p]], gbufs[b], gsems[b])

        def wait_gather(p, b):
            pltpu.make_async_copy(tok_hbm.at[idx_v.at[p]], gbufs[b],
                                  gsems[b]).wait()

        def issue_out(p, t):
            for e in range(EMB // 8):
                pltpu.async_copy(
                    tbufs[t].at[pl.ds(8 * BATCH_LANES * e, 8 * BATCH_LANES)],
                    out_hbm.at[p, e, wid], osems[t])

        def wait_out(p, t):
            for e in range(EMB // 8):
                pltpu.make_async_copy(
                    tbufs[t].at[pl.ds(8 * BATCH_LANES * e, 8 * BATCH_LANES)],
                    out_hbm.at[p, e, wid], osems[t]).wait()

        issue_gather(0, 0)
        issue_gather(1, 1)

        @pl.loop(0, MAXLEN, step=NGBUF)
        def _grp(g):
            for b in range(NGBUF):
                p = g + b
                t = b % 2
                bp = (b + 2) % NGBUF
                wait_gather(p, b)

                @pl.when(p + 2 < MAXLEN)
                def _prefetch():
                    issue_gather(p + 2, bp)

                @pl.when(p >= 2)
                def _drain():
                    wait_out(p - 2, t)

                for c in range(EMB // 16):
                    pos_e = pos_v[p, pl.ds(16 * c, 16)]
                    tgt = (lane + 16 * c) * BATCH_LANES

                    @pl.loop(0, BATCH_LANES, unroll=4)
                    def _row(r):
                        v = gbufs[b][r, pl.ds(16 * c, 16)] + pos_e
                        plsc.store_scatter(tbufs[t], [tgt + r], v)

                issue_out(p, t)

        wait_out(MAXLEN - 2, 0)
        wait_out(MAXLEN - 1, 1)

    return k(x_t, token_table, pos_table)


def kernel(x, token_table, pos_table):
    batch, seq = x.shape
    if seq < MAXLEN:
        x = jnp.pad(x, ((0, 0), (0, MAXLEN - seq)))
    else:
        x = x[:, :MAXLEN]
    x_t = x.T.astype(jnp.int32)
    out4 = _tok_pos_embed(x_t, token_table, pos_table, batch)
    out5 = out4.reshape(MAXLEN, EMB // 8, batch // BATCH_LANES, 8, BATCH_LANES)
    return out5.transpose(2, 4, 0, 1, 3).reshape(batch, MAXLEN, EMB)


# speedup vs baseline: 1.5622x; 1.5622x over previous
import functools

import jax
import jax.numpy as jnp
from jax import lax
from jax.experimental import pallas as pl
from jax.experimental.pallas import tpu as pltpu
from jax.experimental.pallas import tpu_sc as plsc

MAXLEN = 200
EMB = 64
BATCH_LANES = 128
NUM_TILES = 32
NGBUF = 4


def _tok_pos_embed(x_t, token_table, pos_table, batch):
    nb = batch // BATCH_LANES
    assert nb == NUM_TILES
    mesh = plsc.VectorSubcoreMesh(core_axis_name="c", subcore_axis_name="s")

    @functools.partial(
        pl.kernel,
        out_type=jax.ShapeDtypeStruct((MAXLEN, EMB // 8, nb, 8, BATCH_LANES),
                                      jnp.float32),
        mesh=mesh,
        compiler_params=pltpu.CompilerParams(use_tc_tiling_on_sc=False,
                                             needs_layout_passes=False),
        scratch_types=[
            pltpu.VMEM((MAXLEN, BATCH_LANES), jnp.int32),
            pltpu.VMEM((MAXLEN, EMB), jnp.float32),
        ] + [pltpu.VMEM((BATCH_LANES, EMB), jnp.float32) for _ in range(NGBUF)]
          + [pltpu.VMEM((EMB, BATCH_LANES + 1), jnp.float32) for _ in range(2)]
          + [pltpu.SemaphoreType.DMA for _ in range(NGBUF + 2)],
    )
    def k(x_hbm, tok_hbm, pos_hbm, out_hbm, idx_v, pos_v, *bufs_and_sems):
        gbufs = bufs_and_sems[:NGBUF]
        tbufs = bufs_and_sems[NGBUF:NGBUF + 2]
        gsems = bufs_and_sems[NGBUF + 2:2 * NGBUF + 2]
        osems = bufs_and_sems[2 * NGBUF + 2:]
        wid = lax.axis_index("s") * 2 + lax.axis_index("c")
        pltpu.sync_copy(x_hbm.at[:, pl.ds(wid * BATCH_LANES, BATCH_LANES)],
                        idx_v)
        pltpu.sync_copy(pos_hbm, pos_v)
        lane = lax.iota(jnp.int32, 16)

        def issue_gather(p, b):
            pltpu.async_copy(tok_hbm.at[idx_v.at[p]], gbufs[b], gsems[b])

        def wait_gather(p, b):
            pltpu.make_async_copy(tok_hbm.at[idx_v.at[p]], gbufs[b],
                                  gsems[b]).wait()

        def issue_out(p, t):
            for e in range(EMB // 8):
                pltpu.async_copy(
                    tbufs[t].at[pl.ds(8 * e, 8), pl.ds(0, BATCH_LANES)],
                    out_hbm.at[p, e, wid], osems[t])

        def wait_out(p, t):
            for e in range(EMB // 8):
                pltpu.make_async_copy(
                    tbufs[t].at[pl.ds(8 * e, 8), pl.ds(0, BATCH_LANES)],
                    out_hbm.at[p, e, wid], osems[t]).wait()

        issue_gather(0, 0)
        issue_gather(1, 1)

        @pl.loop(0, MAXLEN, step=NGBUF)
        def _grp(g):
            for b in range(NGBUF):
                p = g + b
                t = b % 2
                bp = (b + 2) % NGBUF
                wait_gather(p, b)

                @pl.when(p + 2 < MAXLEN)
                def _prefetch():
                    issue_gather(p + 2, bp)

                @pl.when(p >= 2)
                def _drain():
                    wait_out(p - 2, t)

                pos_es = [pos_v[p, pl.ds(16 * c, 16)] for c in range(EMB // 16)]

                @pl.loop(0, BATCH_LANES, unroll=4)
                def _row(r):
                    rvec = lane * 0 + r
                    for c in range(EMB // 16):
                        v = gbufs[b][r, pl.ds(16 * c, 16)] + pos_es[c]
                        plsc.store_scatter(tbufs[t], [lane + 16 * c, rvec], v)

                issue_out(p, t)

        wait_out(MAXLEN - 2, 0)
        wait_out(MAXLEN - 1, 1)

    return k(x_t, token_table, pos_table)


def kernel(x, token_table, pos_table):
    batch, seq = x.shape
    if seq < MAXLEN:
        x = jnp.pad(x, ((0, 0), (0, MAXLEN - seq)))
    else:
        x = x[:, :MAXLEN]
    x_t = x.T.astype(jnp.int32)
    out4 = _tok_pos_embed(x_t, token_table, pos_table, batch)
    return out4.transpose(2, 4, 0, 1, 3).reshape(batch, MAXLEN, EMB)
